# baseline (speedup 1.0000x reference)
import functools
import math

import jax
import jax.numpy as jnp
from jax import lax
from jax.experimental import pallas as pl
from jax.experimental.pallas import tpu as pltpu
from jax.experimental.pallas import tpu_sc as plsc

D_MODEL = 64
SCALE = math.sqrt(D_MODEL)
TBLK = 128
NBUF = 4
L = 16


def _emb_kernel(idx_hbm, tab_hbm, out_hbm, idx_v, gbufs, sbufs, gsems, ssems,
                *, n_s, nc):
    wid = lax.axis_index("s") * nc + lax.axis_index("c")
    t0 = wid * TBLK
    pltpu.sync_copy(idx_hbm.at[:, pl.ds(t0, TBLK)], idx_v)

    def gather(s, b):
        return pltpu.make_async_copy(
            tab_hbm.at[idx_v.at[s]], gbufs[b], gsems[b])

    def store(s, b):
        return pltpu.make_async_copy(
            sbufs[b], out_hbm.at[s, :, pl.ds(t0, TBLK)], ssems[b])

    row_ids = [lax.iota(jnp.int32, L) + tb * L for tb in range(TBLK // L)]

    for b in range(NBUF):
        gather(b, b).start()

    n_groups = n_s // NBUF

    def group(g, carry):
        for b in range(NBUF):
            s = g * NBUF + b
            gather(s, b).wait()

            @pl.when(g > 0)
            def _wait_prev_store():
                store(s - NBUF, b).wait()

            def tpose_col(d, carry2):
                dvec = jnp.full((L,), 0, jnp.int32) + d
                for tb in range(TBLK // L):
                    v = plsc.load_gather(gbufs[b], [row_ids[tb], dvec])
                    sbufs[b][d, pl.ds(tb * L, L)] = v * SCALE
                return carry2

            lax.fori_loop(0, D_MODEL, tpose_col, 0)
            store(s, b).start()

            @pl.when(s + NBUF < n_s)
            def _fire_next_gather():
                gather(s + NBUF, b).start()

        return carry

    lax.fori_loop(0, n_groups, group, 0)
    for b in range(NBUF):
        store((n_groups - 1) * NBUF + b, b).wait()


def kernel(inputs, table):
    n_tok, seq = inputs.shape
    info = plsc.get_sparse_core_info()
    nc, ns = info.num_cores, info.num_subcores
    nw = nc * ns
    assert n_tok == nw * TBLK and seq % NBUF == 0

    idx_t = inputs.astype(jnp.int32).T

    mesh = plsc.VectorSubcoreMesh(core_axis_name="c", subcore_axis_name="s")
    k = functools.partial(
        pl.kernel,
        out_type=jax.ShapeDtypeStruct((seq, D_MODEL, n_tok), jnp.float32),
        mesh=mesh,
        scratch_types=[
            pltpu.VMEM((seq, TBLK), jnp.int32),
            [pltpu.VMEM((TBLK, D_MODEL), jnp.float32) for _ in range(NBUF)],
            [pltpu.VMEM((D_MODEL, TBLK), jnp.float32) for _ in range(NBUF)],
            [pltpu.SemaphoreType.DMA for _ in range(NBUF)],
            [pltpu.SemaphoreType.DMA for _ in range(NBUF)],
        ],
        compiler_params=pltpu.CompilerParams(use_tc_tiling_on_sc=False,
                                             needs_layout_passes=False),
    )(functools.partial(_emb_kernel, n_s=seq, nc=nc))

    out = k(idx_t, table)
    return jnp.transpose(out, (2, 0, 1))


# speedup vs baseline: 1.1176x; 1.1176x over previous
import functools
import math

import jax
import jax.numpy as jnp
from jax import lax
from jax.experimental import pallas as pl
from jax.experimental.pallas import tpu as pltpu
from jax.experimental.pallas import tpu_sc as plsc

D_MODEL = 64
SCALE = math.sqrt(D_MODEL)
TBLK = 128
NBUF = 4
L = 16


def _emb_kernel(idx_hbm, tab_hbm, out_hbm, idx_v, gbufs, sbufs, gsems, ssems,
                *, n_s, nc):
    wid = lax.axis_index("s") * nc + lax.axis_index("c")
    t0 = wid * TBLK
    pltpu.sync_copy(idx_hbm.at[:, pl.ds(t0, TBLK)], idx_v)

    def gather(s, b):
        return pltpu.make_async_copy(
            tab_hbm.at[idx_v.at[s]], gbufs[b], gsems[b])

    def store(s, b):
        return pltpu.make_async_copy(
            sbufs[b], out_hbm.at[s, :, pl.ds(t0, TBLK)], ssems[b])

    dvec_q = [lax.iota(jnp.int32, L) + q * L for q in range(D_MODEL // L)]

    for b in range(NBUF):
        gather(b, b).start()

    n_groups = n_s // NBUF

    def group(g, carry):
        for b in range(NBUF):
            s = g * NBUF + b
            gather(s, b).wait()

            @pl.when(g > 0)
            def _wait_prev_store():
                store(s - NBUF, b).wait()

            def tpose_row(t, carry2):
                tvec = jnp.full((L,), 0, jnp.int32) + t
                for q in range(D_MODEL // L):
                    v = gbufs[b][t, pl.ds(q * L, L)] * SCALE
                    plsc.store_scatter(sbufs[b], [dvec_q[q], tvec], v)
                return carry2

            lax.fori_loop(0, TBLK, tpose_row, 0, unroll=4)
            store(s, b).start()

            @pl.when(s + NBUF < n_s)
            def _fire_next_gather():
                gather(s + NBUF, b).start()

        return carry

    lax.fori_loop(0, n_groups, group, 0)
    for b in range(NBUF):
        store((n_groups - 1) * NBUF + b, b).wait()


def kernel(inputs, table):
    n_tok, seq = inputs.shape
    info = plsc.get_sparse_core_info()
    nc, ns = info.num_cores, info.num_subcores
    nw = nc * ns
    assert n_tok == nw * TBLK and seq % NBUF == 0

    idx_t = inputs.astype(jnp.int32).T

    mesh = plsc.VectorSubcoreMesh(core_axis_name="c", subcore_axis_name="s")
    k = functools.partial(
        pl.kernel,
        out_type=jax.ShapeDtypeStruct((seq, D_MODEL, n_tok), jnp.float32),
        mesh=mesh,
        scratch_types=[
            pltpu.VMEM((seq, TBLK), jnp.int32),
            [pltpu.VMEM((TBLK, D_MODEL), jnp.float32) for _ in range(NBUF)],
            [pltpu.VMEM((D_MODEL, TBLK), jnp.float32) for _ in range(NBUF)],
            [pltpu.SemaphoreType.DMA for _ in range(NBUF)],
            [pltpu.SemaphoreType.DMA for _ in range(NBUF)],
        ],
        compiler_params=pltpu.CompilerParams(use_tc_tiling_on_sc=False,
                                             needs_layout_passes=False),
    )(functools.partial(_emb_kernel, n_s=seq, nc=nc))

    out = k(idx_t, table)
    return jnp.transpose(out, (2, 0, 1))


# speedup vs baseline: 2.1789x; 1.9496x over previous
import functools
import math

import jax
import jax.numpy as jnp
from jax import lax
from jax.experimental import pallas as pl
from jax.experimental.pallas import tpu as pltpu
from jax.experimental.pallas import tpu_sc as plsc

D_MODEL = 64
D_PAD = 128
SCALE = math.sqrt(D_MODEL)
TBLK = 128
NBUF = 4
L = 16


def _emb_kernel(idx_hbm, tab_hbm, out_hbm, idx_v, gbufs, gsems, ssems,
                *, n_s, nc, nw):
    wid = lax.axis_index("s") * nc + lax.axis_index("c")
    t0 = wid * TBLK
    pltpu.sync_copy(idx_hbm.at[:, pl.ds(t0, TBLK)], idx_v)

    def gather(s, b):
        return pltpu.make_async_copy(
            tab_hbm.at[idx_v.at[s]], gbufs[b], gsems[b])

    def store(s, b):
        return pltpu.make_async_copy(
            gbufs[b], out_hbm.at[s * nw + wid], ssems[b])

    for b in range(NBUF):
        gather(b, b).start()

    n_groups = n_s // NBUF

    def group(g, carry):
        for b in range(NBUF):
            s = g * NBUF + b
            gather(s, b).wait()

            @pl.when(g > 0)
            def _wait_prev_store():
                store(s - NBUF, b).wait()

            def scale_row(t, carry2):
                for q in range(D_MODEL // L):
                    sl = pl.ds(q * L, L)
                    gbufs[b][t, sl] = gbufs[b][t, sl] * SCALE
                return carry2

            lax.fori_loop(0, TBLK, scale_row, 0, unroll=4)
            store(s, b).start()

            @pl.when(s + NBUF < n_s)
            def _fire_next_gather():
                gather(s + NBUF, b).start()

        return carry

    lax.fori_loop(0, n_groups, group, 0)
    for b in range(NBUF):
        store((n_groups - 1) * NBUF + b, b).wait()


def kernel(inputs, table):
    n_tok, seq = inputs.shape
    info = plsc.get_sparse_core_info()
    nc, ns = info.num_cores, info.num_subcores
    nw = nc * ns
    assert n_tok == nw * TBLK and seq % NBUF == 0

    idx_t = inputs.astype(jnp.int32).T
    tab128 = jnp.pad(table, ((0, 0), (0, D_PAD - D_MODEL)))

    mesh = plsc.VectorSubcoreMesh(core_axis_name="c", subcore_axis_name="s")
    k = functools.partial(
        pl.kernel,
        out_type=jax.ShapeDtypeStruct((seq * nw, TBLK, D_PAD), jnp.float32),
        mesh=mesh,
        scratch_types=[
            pltpu.VMEM((seq, TBLK), jnp.int32),
            [pltpu.VMEM((TBLK, D_PAD), jnp.float32) for _ in range(NBUF)],
            [pltpu.SemaphoreType.DMA for _ in range(NBUF)],
            [pltpu.SemaphoreType.DMA for _ in range(NBUF)],
        ],
    )(functools.partial(_emb_kernel, n_s=seq, nc=nc, nw=nw))

    out = k(idx_t, tab128)
    out = out[:, :, :D_MODEL].reshape(seq, nw * TBLK, D_MODEL)
    return jnp.transpose(out, (1, 0, 2))
